# baseline (speedup 1.0000x reference)
import functools
import jax
import jax.numpy as jnp
from jax.experimental import pallas as pl

D = 128


def _mm_body(x_ref, k_ref, o_ref):
    o_ref[...] = jnp.dot(x_ref[...], k_ref[...],
                         preferred_element_type=jnp.float32)


def _dense_mm(x, K, block_rows):
    n = x.shape[0]
    grid = n // block_rows
    return pl.pallas_call(
        _mm_body,
        grid=(grid,),
        in_specs=[pl.BlockSpec((block_rows, x.shape[1]), lambda i: (i, 0)),
                  pl.BlockSpec((x.shape[1], K.shape[1]), lambda i: (0, 0))],
        out_specs=pl.BlockSpec((block_rows, K.shape[1]), lambda i: (i, 0)),
        out_shape=jax.ShapeDtypeStruct((n, K.shape[1]), jnp.float32),
    )(x, K)


def _post_body(p_ref, m_ref, x_ref, w_ref, b_ref, o_ref):
    h = jnp.dot(p_ref[...], m_ref[...], preferred_element_type=jnp.float32)
    mu = jnp.mean(h, -1, keepdims=True)
    var = jnp.mean((h - mu) ** 2, -1, keepdims=True)
    hn = (h - mu) * jax.lax.rsqrt(var + 1e-5) * w_ref[...] + b_ref[...]
    hn = hn + x_ref[...]
    o_ref[...] = jnp.where(hn > 0, hn, jnp.exp(jnp.minimum(hn, 0.0)) - 1.0)


def _post(Pc, M, xres, ln_w, ln_b, block_rows):
    n = Pc.shape[0]
    grid = n // block_rows
    return pl.pallas_call(
        _post_body,
        grid=(grid,),
        in_specs=[pl.BlockSpec((block_rows, Pc.shape[1]), lambda i: (i, 0)),
                  pl.BlockSpec((Pc.shape[1], D), lambda i: (0, 0)),
                  pl.BlockSpec((block_rows, D), lambda i: (i, 0)),
                  pl.BlockSpec((1, D), lambda i: (0, 0)),
                  pl.BlockSpec((1, D), lambda i: (0, 0))],
        out_specs=pl.BlockSpec((block_rows, D), lambda i: (i, 0)),
        out_shape=jax.ShapeDtypeStruct((n, D), jnp.float32),
    )(Pc, M, xres, ln_w.reshape(1, D), ln_b.reshape(1, D))


def _edge_pass(e_logit, dst, n_dst):
    ex = jnp.exp(e_logit)
    s = jax.ops.segment_sum(ex, dst, num_segments=n_dst)
    return ex / (s[dst] + 1e-16)


def kernel(x_user, x_product, x_category, x_brand, edge_index_view,
           edge_index_cart, edge_index_purchase, edge_index_rev_purchase,
           edge_index_belongs_to, edge_attr_belongs_to, W_base, A, B,
           rel_emb, beh_emb, a_att, ln_w, ln_b):
    d = D
    a0, a1, a2, a3 = a_att[:d], a_att[d:2*d], a_att[2*d:3*d], a_att[3*d:]
    W0 = W_base + A[0] @ B[0].T
    W1 = W_base + A[0] @ B[1].T
    W2 = W_base + A[0] @ B[2].T
    W3 = W_base + A[1] @ B[2].T

    def cconst(rel, beta):
        return (rel_emb[rel] * a2).sum() + (beh_emb[beta] * a3).sum()

    q1 = a0 @ A[1]
    Ku = jnp.zeros((d, 128))
    Ku = Ku.at[:, 0].set(a0 @ W0).at[:, 1].set(a0 @ W1).at[:, 2].set(a0 @ W2)
    Ku = Ku.at[:, 3].set(a1 @ W3)
    Ku = Ku.at[:, 8:56].set(jnp.concatenate([B[0], B[1], B[2]], 1))
    Kp = jnp.zeros((d, 128))
    Kp = Kp.at[:, 0].set(a1 @ W0).at[:, 1].set(a1 @ W1).at[:, 2].set(a1 @ W2)
    Kp = Kp.at[:, 3].set(a0 @ W3)
    for o in range(3):
        Kp = Kp.at[:, 4 + o].set((a0 @ W_base) + B[o] @ q1)
    Kp = Kp.at[:, 8:56].set(jnp.concatenate([B[0], B[1], B[2]], 1))

    Fu = _dense_mm(x_user, Ku, 1000)
    Fp = _dense_mm(x_product, Kp, 1000)
    v_cat = x_category @ (a1 @ W_base)

    Nu, Np_, Nc = x_user.shape[0], x_product.shape[0], x_category.shape[0]
    cu = jnp.array([cconst(0, 0), cconst(1, 1), cconst(2, 2)])
    c3 = cconst(5, 2)
    c4 = jnp.array([cconst(6, o) for o in range(3)])

    P128p = jnp.zeros((Np_, 128))
    P16p = jnp.zeros((Np_, 16))
    for t, ei in enumerate([edge_index_view, edge_index_cart,
                            edge_index_purchase]):
        src, dst = ei[0], ei[1]
        e = Fu[src, t] + Fp[dst, t] + cu[t]
        al = _edge_pass(e, dst, Np_)
        P128p = P128p.at[dst].add(al[:, None] * x_user[src])
        P16p = P16p.at[dst].add(al[:, None] * Fu[src, 8+16*t:8+16*(t+1)])

    src, dst = edge_index_rev_purchase[0], edge_index_rev_purchase[1]
    e = Fp[src, 3] + Fu[dst, 3] + c3
    al = _edge_pass(e, dst, Nu)
    P128u = jnp.zeros((Nu, 128)).at[dst].add(al[:, None] * x_product[src])
    P16u = jnp.zeros((Nu, 16)).at[dst].add(al[:, None] * Fp[src, 40:56])

    src, dst = edge_index_belongs_to[0], edge_index_belongs_to[1]
    o = jnp.clip(edge_attr_belongs_to.reshape(-1), 0, 2)
    e = Fp[src, 4 + o] + v_cat[dst] + c4[o]
    al = _edge_pass(e, dst, Nc)
    P128c = jnp.zeros((Nc, 128)).at[dst].add(al[:, None] * x_product[src])
    zo = Fp[:, 8:56].reshape(Np_, 3, 16)[src, o]
    P16c = jnp.zeros((Nc, 16)).at[dst].add(al[:, None] * zo)

    def mfor(Aphi):
        return jnp.concatenate(
            [W_base.T, Aphi.T, jnp.zeros((128 - 16, D))], 0)

    Mu_ = mfor(A[1])
    Mp_ = mfor(A[0])
    pad = lambda P16, n: jnp.concatenate([P16, jnp.zeros((n, 112))], 1)
    Pcu = jnp.concatenate([P128u, pad(P16u, Nu)], 1)
    Pcp = jnp.concatenate([P128p, pad(P16p, Np_)], 1)
    Pcc = jnp.concatenate([P128c, pad(P16c, Nc)], 1)
    out_u = _post(Pcu, Mu_, x_user, ln_w, ln_b, 1000)
    out_p = _post(Pcp, Mp_, x_product, ln_w, ln_b, 1000)
    out_c = _post(Pcc, Mu_, x_category, ln_w, ln_b, 1000)
    return out_u, out_p, out_c, x_brand


# speedup vs baseline: 1.0200x; 1.0200x over previous
import functools
import jax
import jax.numpy as jnp
from jax.experimental import pallas as pl

D = 128


def _mm_body(x_ref, k_ref, o_ref):
    o_ref[...] = jnp.dot(x_ref[...], k_ref[...],
                         preferred_element_type=jnp.float32)


def _dense_mm(x, K, block_rows):
    n = x.shape[0]
    grid = n // block_rows
    return pl.pallas_call(
        _mm_body,
        grid=(grid,),
        in_specs=[pl.BlockSpec((block_rows, x.shape[1]), lambda i: (i, 0)),
                  pl.BlockSpec((x.shape[1], K.shape[1]), lambda i: (0, 0))],
        out_specs=pl.BlockSpec((block_rows, K.shape[1]), lambda i: (i, 0)),
        out_shape=jax.ShapeDtypeStruct((n, K.shape[1]), jnp.float32),
    )(x, K)


def _post_body(p_ref, m_ref, x_ref, w_ref, b_ref, o_ref):
    h = jnp.dot(p_ref[...], m_ref[...], preferred_element_type=jnp.float32)
    mu = jnp.mean(h, -1, keepdims=True)
    var = jnp.mean((h - mu) ** 2, -1, keepdims=True)
    hn = (h - mu) * jax.lax.rsqrt(var + 1e-5) * w_ref[...] + b_ref[...]
    hn = hn + x_ref[...]
    o_ref[...] = jnp.where(hn > 0, hn, jnp.exp(jnp.minimum(hn, 0.0)) - 1.0)


def _post(Pc, M, xres, ln_w, ln_b, block_rows):
    n = Pc.shape[0]
    grid = n // block_rows
    return pl.pallas_call(
        _post_body,
        grid=(grid,),
        in_specs=[pl.BlockSpec((block_rows, Pc.shape[1]), lambda i: (i, 0)),
                  pl.BlockSpec((Pc.shape[1], D), lambda i: (0, 0)),
                  pl.BlockSpec((block_rows, D), lambda i: (i, 0)),
                  pl.BlockSpec((1, D), lambda i: (0, 0)),
                  pl.BlockSpec((1, D), lambda i: (0, 0))],
        out_specs=pl.BlockSpec((block_rows, D), lambda i: (i, 0)),
        out_shape=jax.ShapeDtypeStruct((n, D), jnp.float32),
    )(Pc, M, xres, ln_w.reshape(1, D), ln_b.reshape(1, D))


def _edge_pass(e_logit, dst, n_dst):
    ex = jnp.exp(e_logit)
    s = jax.ops.segment_sum(ex, dst, num_segments=n_dst)
    return ex / (s[dst] + 1e-16)


def kernel(x_user, x_product, x_category, x_brand, edge_index_view,
           edge_index_cart, edge_index_purchase, edge_index_rev_purchase,
           edge_index_belongs_to, edge_attr_belongs_to, W_base, A, B,
           rel_emb, beh_emb, a_att, ln_w, ln_b):
    d = D
    a0, a1, a2, a3 = a_att[:d], a_att[d:2*d], a_att[2*d:3*d], a_att[3*d:]
    W0 = W_base + A[0] @ B[0].T
    W1 = W_base + A[0] @ B[1].T
    W2 = W_base + A[0] @ B[2].T
    W3 = W_base + A[1] @ B[2].T

    def cconst(rel, beta):
        return (rel_emb[rel] * a2).sum() + (beh_emb[beta] * a3).sum()

    q1 = a0 @ A[1]
    Ku = jnp.zeros((d, 128))
    Ku = Ku.at[:, 0].set(a0 @ W0).at[:, 1].set(a0 @ W1).at[:, 2].set(a0 @ W2)
    Ku = Ku.at[:, 3].set(a1 @ W3)
    Ku = Ku.at[:, 8:56].set(jnp.concatenate([B[0], B[1], B[2]], 1))
    Kp = jnp.zeros((d, 128))
    Kp = Kp.at[:, 0].set(a1 @ W0).at[:, 1].set(a1 @ W1).at[:, 2].set(a1 @ W2)
    Kp = Kp.at[:, 3].set(a0 @ W3)
    for o in range(3):
        Kp = Kp.at[:, 4 + o].set((a0 @ W_base) + B[o] @ q1)
    Kp = Kp.at[:, 8:56].set(jnp.concatenate([B[0], B[1], B[2]], 1))

    Fu = _dense_mm(x_user, Ku, 1000)
    Fp = _dense_mm(x_product, Kp, 1000)
    v_cat = x_category @ (a1 @ W_base)

    Nu, Np_, Nc = x_user.shape[0], x_product.shape[0], x_category.shape[0]
    cu = jnp.array([cconst(0, 0), cconst(1, 1), cconst(2, 2)])
    c3 = cconst(5, 2)
    c4 = jnp.array([cconst(6, o) for o in range(3)])

    P128p = jnp.zeros((Np_, 128))
    P16p = jnp.zeros((Np_, 16))
    for t, ei in enumerate([edge_index_view, edge_index_cart,
                            edge_index_purchase]):
        src, dst = ei[0], ei[1]
        e = Fu[src, t] + Fp[dst, t] + cu[t]
        al = _edge_pass(e, dst, Np_)
        P128p = P128p + jax.ops.segment_sum(al[:, None] * x_user[src], dst,
                                            num_segments=Np_)
        P16p = P16p + jax.ops.segment_sum(
            al[:, None] * Fu[src, 8+16*t:8+16*(t+1)], dst, num_segments=Np_)

    src, dst = edge_index_rev_purchase[0], edge_index_rev_purchase[1]
    e = Fp[src, 3] + Fu[dst, 3] + c3
    al = _edge_pass(e, dst, Nu)
    P128u = jax.ops.segment_sum(al[:, None] * x_product[src], dst,
                                num_segments=Nu)
    P16u = jax.ops.segment_sum(al[:, None] * Fp[src, 40:56], dst,
                               num_segments=Nu)

    src, dst = edge_index_belongs_to[0], edge_index_belongs_to[1]
    o = jnp.clip(edge_attr_belongs_to.reshape(-1), 0, 2)
    e = Fp[src, 4 + o] + v_cat[dst] + c4[o]
    al = _edge_pass(e, dst, Nc)
    P128c = jax.ops.segment_sum(al[:, None] * x_product[src], dst,
                                num_segments=Nc)
    zo = Fp[:, 8:56].reshape(Np_, 3, 16)[src, o]
    P16c = jax.ops.segment_sum(al[:, None] * zo, dst, num_segments=Nc)

    def mfor(Aphi):
        return jnp.concatenate(
            [W_base.T, Aphi.T, jnp.zeros((128 - 16, D))], 0)

    Mu_ = mfor(A[1])
    Mp_ = mfor(A[0])
    pad = lambda P16, n: jnp.concatenate([P16, jnp.zeros((n, 112))], 1)
    Pcu = jnp.concatenate([P128u, pad(P16u, Nu)], 1)
    Pcp = jnp.concatenate([P128p, pad(P16p, Np_)], 1)
    Pcc = jnp.concatenate([P128c, pad(P16c, Nc)], 1)
    out_u = _post(Pcu, Mu_, x_user, ln_w, ln_b, 1000)
    out_p = _post(Pcp, Mp_, x_product, ln_w, ln_b, 1000)
    out_c = _post(Pcc, Mu_, x_category, ln_w, ln_b, 1000)
    return out_u, out_p, out_c, x_brand


# speedup vs baseline: 24.6442x; 24.1616x over previous
import functools
import jax
import jax.numpy as jnp
from jax.experimental import pallas as pl

D = 128


def _mm_body(x_ref, k_ref, o_ref):
    o_ref[...] = jnp.dot(x_ref[...], k_ref[...],
                         preferred_element_type=jnp.float32)


def _dense_mm(x, K, block_rows):
    n = x.shape[0]
    grid = n // block_rows
    return pl.pallas_call(
        _mm_body,
        grid=(grid,),
        in_specs=[pl.BlockSpec((block_rows, x.shape[1]), lambda i: (i, 0)),
                  pl.BlockSpec((x.shape[1], K.shape[1]), lambda i: (0, 0))],
        out_specs=pl.BlockSpec((block_rows, K.shape[1]), lambda i: (i, 0)),
        out_shape=jax.ShapeDtypeStruct((n, K.shape[1]), jnp.float32),
    )(x, K)


def _post_body(p_ref, m_ref, x_ref, w_ref, b_ref, o_ref):
    h = jnp.dot(p_ref[...], m_ref[...], preferred_element_type=jnp.float32)
    mu = jnp.mean(h, -1, keepdims=True)
    var = jnp.mean((h - mu) ** 2, -1, keepdims=True)
    hn = (h - mu) * jax.lax.rsqrt(var + 1e-5) * w_ref[...] + b_ref[...]
    hn = hn + x_ref[...]
    o_ref[...] = jnp.where(hn > 0, hn, jnp.exp(jnp.minimum(hn, 0.0)) - 1.0)


def _post(Pc, M, xres, ln_w, ln_b, block_rows):
    n = Pc.shape[0]
    grid = n // block_rows
    return pl.pallas_call(
        _post_body,
        grid=(grid,),
        in_specs=[pl.BlockSpec((block_rows, Pc.shape[1]), lambda i: (i, 0)),
                  pl.BlockSpec((Pc.shape[1], D), lambda i: (0, 0)),
                  pl.BlockSpec((block_rows, D), lambda i: (i, 0)),
                  pl.BlockSpec((1, D), lambda i: (0, 0)),
                  pl.BlockSpec((1, D), lambda i: (0, 0))],
        out_specs=pl.BlockSpec((block_rows, D), lambda i: (i, 0)),
        out_shape=jax.ShapeDtypeStruct((n, D), jnp.float32),
    )(Pc, M, xres, ln_w.reshape(1, D), ln_b.reshape(1, D))


def _edge_pass(e_logit, dst, n_dst):
    ex = jnp.exp(e_logit)
    s = jax.ops.segment_sum(ex, dst, num_segments=n_dst)
    return ex / (s[dst] + 1e-16)


def kernel(x_user, x_product, x_category, x_brand, edge_index_view,
           edge_index_cart, edge_index_purchase, edge_index_rev_purchase,
           edge_index_belongs_to, edge_attr_belongs_to, W_base, A, B,
           rel_emb, beh_emb, a_att, ln_w, ln_b):
    d = D
    a0, a1, a2, a3 = a_att[:d], a_att[d:2*d], a_att[2*d:3*d], a_att[3*d:]
    W0 = W_base + A[0] @ B[0].T
    W1 = W_base + A[0] @ B[1].T
    W2 = W_base + A[0] @ B[2].T
    W3 = W_base + A[1] @ B[2].T

    def cconst(rel, beta):
        return (rel_emb[rel] * a2).sum() + (beh_emb[beta] * a3).sum()

    q1 = a0 @ A[1]
    Ku = jnp.zeros((d, 128))
    Ku = Ku.at[:, 0].set(a0 @ W0).at[:, 1].set(a0 @ W1).at[:, 2].set(a0 @ W2)
    Ku = Ku.at[:, 3].set(a1 @ W3)
    Ku = Ku.at[:, 8:56].set(jnp.concatenate([B[0], B[1], B[2]], 1))
    Kp = jnp.zeros((d, 128))
    Kp = Kp.at[:, 0].set(a1 @ W0).at[:, 1].set(a1 @ W1).at[:, 2].set(a1 @ W2)
    Kp = Kp.at[:, 3].set(a0 @ W3)
    for o in range(3):
        Kp = Kp.at[:, 4 + o].set((a0 @ W_base) + B[o] @ q1)
    Kp = Kp.at[:, 8:56].set(jnp.concatenate([B[0], B[1], B[2]], 1))

    Fu = _dense_mm(x_user, Ku, 1000)
    Fp = _dense_mm(x_product, Kp, 1000)
    v_cat = x_category @ (a1 @ W_base)

    Nu, Np_, Nc = x_user.shape[0], x_product.shape[0], x_category.shape[0]
    cu = jnp.array([cconst(0, 0), cconst(1, 1), cconst(2, 2)])
    c3 = cconst(5, 2)
    c4 = jnp.array([cconst(6, o) for o in range(3)])

    Au = Fu[:, 0:8]
    Zu = Fu[:, 8:56]
    Ap = Fp[:, 0:8]
    Zp = Fp[:, 8:56]

    P128p = jnp.zeros((Np_, 128))
    P16p = jnp.zeros((Np_, 16))
    for t, ei in enumerate([edge_index_view, edge_index_cart,
                            edge_index_purchase]):
        src, dst = ei[0], ei[1]
        gs = Au[src]
        gd = Ap[dst]
        e = gs[:, t] + gd[:, t] + cu[t]
        al = _edge_pass(e, dst, Np_)
        P128p = P128p + jax.ops.segment_sum(al[:, None] * x_user[src], dst,
                                            num_segments=Np_)
        P16p = P16p + jax.ops.segment_sum(
            al[:, None] * Zu[src][:, 16*t:16*(t+1)], dst, num_segments=Np_)

    src, dst = edge_index_rev_purchase[0], edge_index_rev_purchase[1]
    e = Ap[src][:, 3] + Au[dst][:, 3] + c3
    al = _edge_pass(e, dst, Nu)
    P128u = jax.ops.segment_sum(al[:, None] * x_product[src], dst,
                                num_segments=Nu)
    P16u = jax.ops.segment_sum(al[:, None] * Zp[src][:, 32:48], dst,
                               num_segments=Nu)

    src, dst = edge_index_belongs_to[0], edge_index_belongs_to[1]
    o = jnp.clip(edge_attr_belongs_to.reshape(-1), 0, 2)
    m0 = (o == 0).astype(jnp.float32)[:, None]
    m1 = (o == 1).astype(jnp.float32)[:, None]
    m2 = (o == 2).astype(jnp.float32)[:, None]
    gs = Ap[src]
    e = (gs[:, 4] * m0[:, 0] + gs[:, 5] * m1[:, 0] + gs[:, 6] * m2[:, 0]
         + v_cat[dst]
         + c4[0] * m0[:, 0] + c4[1] * m1[:, 0] + c4[2] * m2[:, 0])
    al = _edge_pass(e, dst, Nc)
    P128c = jax.ops.segment_sum(al[:, None] * x_product[src], dst,
                                num_segments=Nc)
    Z = Zp[src]
    zo = Z[:, 0:16] * m0 + Z[:, 16:32] * m1 + Z[:, 32:48] * m2
    P16c = jax.ops.segment_sum(al[:, None] * zo, dst, num_segments=Nc)

    def mfor(Aphi):
        return jnp.concatenate(
            [W_base.T, Aphi.T, jnp.zeros((128 - 16, D))], 0)

    Mu_ = mfor(A[1])
    Mp_ = mfor(A[0])
    pad = lambda P16, n: jnp.concatenate([P16, jnp.zeros((n, 112))], 1)
    Pcu = jnp.concatenate([P128u, pad(P16u, Nu)], 1)
    Pcp = jnp.concatenate([P128p, pad(P16p, Np_)], 1)
    Pcc = jnp.concatenate([P128c, pad(P16c, Nc)], 1)
    out_u = _post(Pcu, Mu_, x_user, ln_w, ln_b, 1000)
    out_p = _post(Pcp, Mp_, x_product, ln_w, ln_b, 1000)
    out_c = _post(Pcc, Mu_, x_category, ln_w, ln_b, 1000)
    return out_u, out_p, out_c, x_brand


# speedup vs baseline: 28.6550x; 1.1627x over previous
import functools
import jax
import jax.numpy as jnp
from jax.experimental import pallas as pl

D = 128


def _mm_body(x_ref, k_ref, o_ref):
    o_ref[...] = jnp.dot(x_ref[...], k_ref[...],
                         preferred_element_type=jnp.float32)


def _dense_mm(x, K, block_rows):
    n = x.shape[0]
    grid = n // block_rows
    return pl.pallas_call(
        _mm_body,
        grid=(grid,),
        in_specs=[pl.BlockSpec((block_rows, x.shape[1]), lambda i: (i, 0)),
                  pl.BlockSpec((x.shape[1], K.shape[1]), lambda i: (0, 0))],
        out_specs=pl.BlockSpec((block_rows, K.shape[1]), lambda i: (i, 0)),
        out_shape=jax.ShapeDtypeStruct((n, K.shape[1]), jnp.float32),
    )(x, K)


def _post_body(p_ref, m_ref, x_ref, w_ref, b_ref, o_ref):
    h = jnp.dot(p_ref[...], m_ref[...], preferred_element_type=jnp.float32)
    mu = jnp.mean(h, -1, keepdims=True)
    var = jnp.mean((h - mu) ** 2, -1, keepdims=True)
    hn = (h - mu) * jax.lax.rsqrt(var + 1e-5) * w_ref[...] + b_ref[...]
    hn = hn + x_ref[...]
    o_ref[...] = jnp.where(hn > 0, hn, jnp.exp(jnp.minimum(hn, 0.0)) - 1.0)


def _post(Pc, M, xres, ln_w, ln_b, block_rows):
    n = Pc.shape[0]
    grid = n // block_rows
    return pl.pallas_call(
        _post_body,
        grid=(grid,),
        in_specs=[pl.BlockSpec((block_rows, Pc.shape[1]), lambda i: (i, 0)),
                  pl.BlockSpec((Pc.shape[1], D), lambda i: (0, 0)),
                  pl.BlockSpec((block_rows, D), lambda i: (i, 0)),
                  pl.BlockSpec((1, D), lambda i: (0, 0)),
                  pl.BlockSpec((1, D), lambda i: (0, 0))],
        out_specs=pl.BlockSpec((block_rows, D), lambda i: (i, 0)),
        out_shape=jax.ShapeDtypeStruct((n, D), jnp.float32),
    )(Pc, M, xres, ln_w.reshape(1, D), ln_b.reshape(1, D))


def _edge_pass(e_logit, dst, n_dst):
    ex = jnp.exp(e_logit)
    s = jax.ops.segment_sum(ex, dst, num_segments=n_dst)
    return ex / (s[dst] + 1e-16)


def kernel(x_user, x_product, x_category, x_brand, edge_index_view,
           edge_index_cart, edge_index_purchase, edge_index_rev_purchase,
           edge_index_belongs_to, edge_attr_belongs_to, W_base, A, B,
           rel_emb, beh_emb, a_att, ln_w, ln_b):
    d = D
    a0, a1, a2, a3 = a_att[:d], a_att[d:2*d], a_att[2*d:3*d], a_att[3*d:]
    W0 = W_base + A[0] @ B[0].T
    W1 = W_base + A[0] @ B[1].T
    W2 = W_base + A[0] @ B[2].T
    W3 = W_base + A[1] @ B[2].T

    def cconst(rel, beta):
        return (rel_emb[rel] * a2).sum() + (beh_emb[beta] * a3).sum()

    q1 = a0 @ A[1]
    Ku = jnp.zeros((d, 128))
    Ku = Ku.at[:, 0].set(a0 @ W0).at[:, 1].set(a0 @ W1).at[:, 2].set(a0 @ W2)
    Ku = Ku.at[:, 3].set(a1 @ W3)
    Ku = Ku.at[:, 8:56].set(jnp.concatenate([B[0], B[1], B[2]], 1))
    Kp = jnp.zeros((d, 128))
    Kp = Kp.at[:, 0].set(a1 @ W0).at[:, 1].set(a1 @ W1).at[:, 2].set(a1 @ W2)
    Kp = Kp.at[:, 3].set(a0 @ W3)
    for o in range(3):
        Kp = Kp.at[:, 4 + o].set((a0 @ W_base) + B[o] @ q1)
    Kp = Kp.at[:, 8:56].set(jnp.concatenate([B[0], B[1], B[2]], 1))

    Fu = _dense_mm(x_user, Ku, 1000)
    Fp = _dense_mm(x_product, Kp, 1000)
    v_cat = x_category @ (a1 @ W_base)

    Nu, Np_, Nc = x_user.shape[0], x_product.shape[0], x_category.shape[0]
    cu = jnp.array([cconst(0, 0), cconst(1, 1), cconst(2, 2)])
    c3 = cconst(5, 2)
    c4 = jnp.array([cconst(6, o) for o in range(3)])

    Au = Fu[:, 0:8]
    Zu = Fu[:, 8:56]
    Ap = Fp[:, 0:8]
    Zp = Fp[:, 8:56]

    P128p = jnp.zeros((Np_, 128))
    P16p = jnp.zeros((Np_, 16))
    for t, ei in enumerate([edge_index_view, edge_index_cart,
                            edge_index_purchase]):
        src, dst = ei[0], ei[1]
        gs = Au[src]
        gd = Ap[dst]
        e = gs[:, t] + gd[:, t] + cu[t]
        al = _edge_pass(e, dst, Np_)
        pay = jnp.concatenate([x_user[src], Zu[src][:, 16*t:16*(t+1)]], 1)
        acc = jax.ops.segment_sum(al[:, None] * pay, dst, num_segments=Np_)
        P128p = P128p + acc[:, :128]
        P16p = P16p + acc[:, 128:]

    src, dst = edge_index_rev_purchase[0], edge_index_rev_purchase[1]
    e = Ap[src][:, 3] + Au[dst][:, 3] + c3
    al = _edge_pass(e, dst, Nu)
    pay = jnp.concatenate([x_product[src], Zp[src][:, 32:48]], 1)
    accu = jax.ops.segment_sum(al[:, None] * pay, dst, num_segments=Nu)
    P128u, P16u = accu[:, :128], accu[:, 128:]

    src, dst = edge_index_belongs_to[0], edge_index_belongs_to[1]
    o = jnp.clip(edge_attr_belongs_to.reshape(-1), 0, 2)
    m0 = (o == 0).astype(jnp.float32)[:, None]
    m1 = (o == 1).astype(jnp.float32)[:, None]
    m2 = (o == 2).astype(jnp.float32)[:, None]
    gs = Ap[src]
    e = (gs[:, 4] * m0[:, 0] + gs[:, 5] * m1[:, 0] + gs[:, 6] * m2[:, 0]
         + v_cat[dst]
         + c4[0] * m0[:, 0] + c4[1] * m1[:, 0] + c4[2] * m2[:, 0])
    al = _edge_pass(e, dst, Nc)
    Z = Zp[src]
    zo = Z[:, 0:16] * m0 + Z[:, 16:32] * m1 + Z[:, 32:48] * m2
    pay = jnp.concatenate([x_product[src], zo], 1)
    accc = jax.ops.segment_sum(al[:, None] * pay, dst, num_segments=Nc)
    P128c, P16c = accc[:, :128], accc[:, 128:]

    def mfor(Aphi):
        return jnp.concatenate(
            [W_base.T, Aphi.T, jnp.zeros((128 - 16, D))], 0)

    Mu_ = mfor(A[1])
    Mp_ = mfor(A[0])
    pad = lambda P16, n: jnp.concatenate([P16, jnp.zeros((n, 112))], 1)
    Pcu = jnp.concatenate([P128u, pad(P16u, Nu)], 1)
    Pcp = jnp.concatenate([P128p, pad(P16p, Np_)], 1)
    Pcc = jnp.concatenate([P128c, pad(P16c, Nc)], 1)
    out_u = _post(Pcu, Mu_, x_user, ln_w, ln_b, 1000)
    out_p = _post(Pcp, Mp_, x_product, ln_w, ln_b, 1000)
    out_c = _post(Pcc, Mu_, x_category, ln_w, ln_b, 1000)
    return out_u, out_p, out_c, x_brand
